# baseline (speedup 1.0000x reference)
import jax
import jax.numpy as jnp
from jax.experimental import pallas as pl

_BM = 2048
_NPAD = 2560


def _cdist_kernel(x_ref, w_ref, out_ref):
    x = x_ref[...]
    w = w_ref[...]
    x_sq = jnp.sum(x * x, axis=1, keepdims=True)
    w_sq = jnp.sum(w * w, axis=1)[None, :]
    xw = jax.lax.dot_general(
        x.astype(jnp.bfloat16),
        (-2.0 * w).astype(jnp.bfloat16),
        (((1,), (1,)), ((), ())),
        preferred_element_type=jnp.float32,
    )
    d2 = jnp.maximum((xw + x_sq) + w_sq, 1e-12)
    out_ref[...] = d2 * jax.lax.rsqrt(d2)


def kernel(x, weights):
    b, k = x.shape
    n = weights.shape[0]
    wp = jnp.pad(weights, ((0, _NPAD - n), (0, 0)))
    out = pl.pallas_call(
        _cdist_kernel,
        grid=(b // _BM,),
        in_specs=[
            pl.BlockSpec((_BM, k), lambda i: (i, 0)),
            pl.BlockSpec((_NPAD, k), lambda i: (0, 0)),
        ],
        out_specs=pl.BlockSpec((_BM, _NPAD), lambda i: (i, 0)),
        out_shape=jax.ShapeDtypeStruct((b, _NPAD), jnp.float32),
    )(x, wp)
    return out[:, :n]


# speedup vs baseline: 1.0056x; 1.0056x over previous
import jax
import jax.numpy as jnp
from jax.experimental import pallas as pl

_BM = 2048
_NPAD = 2560


def _cdist_kernel(x_ref, wt_ref, out_ref):
    x = x_ref[...]
    wt = wt_ref[...]
    x_sq = jnp.sum(x * x, axis=1, keepdims=True)
    w_sq = jnp.sum(wt * wt, axis=0, keepdims=True)
    xw = jnp.dot(
        x.astype(jnp.bfloat16),
        (-2.0 * wt).astype(jnp.bfloat16),
        preferred_element_type=jnp.float32,
    )
    d2 = jnp.maximum((xw + x_sq) + w_sq, 1e-12)
    out_ref[...] = d2 * jax.lax.rsqrt(d2)


def kernel(x, weights):
    b, k = x.shape
    n = weights.shape[0]
    wt = jnp.pad(weights, ((0, _NPAD - n), (0, 0))).T
    out = pl.pallas_call(
        _cdist_kernel,
        grid=(b // _BM,),
        in_specs=[
            pl.BlockSpec((_BM, k), lambda i: (i, 0)),
            pl.BlockSpec((k, _NPAD), lambda i: (0, 0)),
        ],
        out_specs=pl.BlockSpec((_BM, _NPAD), lambda i: (i, 0)),
        out_shape=jax.ShapeDtypeStruct((b, _NPAD), jnp.float32),
    )(x, wt)
    return out[:, :n]
